# Initial kernel scaffold; baseline (speedup 1.0000x reference)
import jax
import jax.numpy as jnp
from jax.experimental import pallas as pl


def kernel(rgb, W1, b1, g1, be1, W2, b2, g2, be2, W3, b3, P1, pb1, P2, pb2):
    raise NotImplementedError("write your pallas kernel here")


# speedup vs baseline: 2.1993x; 2.1993x over previous
import functools

import jax
import jax.numpy as jnp
from jax.experimental import pallas as pl
from jax.experimental.pallas import tpu as pltpu

OUT_D = 256
BLOCK_T = 1024


def _ln(x, g, b, eps=1e-5):
    mu = jnp.mean(x, axis=-1, keepdims=True)
    xc = x - mu
    var = jnp.mean(xc * xc, axis=-1, keepdims=True)
    return xc * jax.lax.rsqrt(var + eps) * g + b


def _gelu(x):
    return 0.5 * x * (1.0 + jax.lax.erf(x * 0.7071067811865476))


def _encoder_kernel(x_ref, W1a_ref, W1b_ref, b1_ref, g1_ref, be1_ref,
                    W2_ref, b2_ref, g2_ref, be2_ref, W3_ref, b3_ref,
                    P1_ref, pb1_ref, P2_ref, pb2_ref,
                    tok_ref, pos_ref):
    x = x_ref[...] * 2.0 - 1.0
    pos_ref[...] = x
    nrm = jnp.sqrt(jnp.sum(x * x, axis=-1, keepdims=True))
    n = x * (1.0 / (nrm + 1e-6))
    h = (jnp.dot(x, W1a_ref[...], preferred_element_type=jnp.float32)
         + jnp.dot(n, W1b_ref[...], preferred_element_type=jnp.float32)
         + b1_ref[...])
    h = _gelu(_ln(h, g1_ref[...], be1_ref[...]))
    h = jnp.dot(h, W2_ref[...], preferred_element_type=jnp.float32) + b2_ref[...]
    h = _gelu(_ln(h, g2_ref[...], be2_ref[...]))
    t = jnp.dot(h, W3_ref[...], preferred_element_type=jnp.float32) + b3_ref[...]
    p = _gelu(jnp.dot(x, P1_ref[...], preferred_element_type=jnp.float32)
              + pb1_ref[...])
    t = t + jnp.dot(p, P2_ref[...], preferred_element_type=jnp.float32) + pb2_ref[...]
    tok_ref[...] = t


def _full(shape):
    return pl.BlockSpec(shape, lambda i: (0,) * len(shape))


@jax.jit
def kernel(rgb, W1, b1, g1, be1, W2, b2, g2, be2, W3, b3, P1, pb1, P2, pb2):
    B, C, H, W = rgb.shape
    N = B * H * W
    x = jnp.transpose(rgb, (0, 2, 3, 1)).reshape(N, 3)

    ws = [W1[:3], W1[3:], b1.reshape(1, -1), g1.reshape(1, -1),
          be1.reshape(1, -1), W2, b2.reshape(1, -1), g2.reshape(1, -1),
          be2.reshape(1, -1), W3, b3.reshape(1, -1), P1,
          pb1.reshape(1, -1), P2, pb2.reshape(1, -1)]

    tok, pos = pl.pallas_call(
        _encoder_kernel,
        grid=(N // BLOCK_T,),
        in_specs=[pl.BlockSpec((BLOCK_T, 3), lambda i: (i, 0))]
                 + [_full(w.shape) for w in ws],
        out_specs=[
            pl.BlockSpec((BLOCK_T, OUT_D), lambda i: (i, 0)),
            pl.BlockSpec((BLOCK_T, 3), lambda i: (i, 0)),
        ],
        out_shape=[
            jax.ShapeDtypeStruct((N, OUT_D), jnp.float32),
            jax.ShapeDtypeStruct((N, 3), jnp.float32),
        ],
        compiler_params=pltpu.CompilerParams(
            dimension_semantics=("parallel",),
        ),
    )(x, *ws)

    tokens = tok.reshape(B, H * W, OUT_D)
    positions = pos.reshape(B, H * W, 3)
    return tokens, positions


# speedup vs baseline: 2.3077x; 1.0493x over previous
import jax
import jax.numpy as jnp
from jax.experimental import pallas as pl
from jax.experimental.pallas import tpu as pltpu

OUT_D = 256
BLOCK_T = 1024


def _gelu(x):
    return 0.5 * x * (1.0 + jax.lax.erf(x * 0.7071067811865476))


def _encoder_kernel(x_ref, WX_ref, b1_ref, g1_ref, be1_ref,
                    W2_ref, b2_ref, g2_ref, be2_ref, W3_ref, b3p_ref,
                    pb1_ref, P2_ref, ones3_ref, J64_ref, J128_ref,
                    tok_ref, pos_ref):
    x = x_ref[...] * 2.0 - 1.0
    pos_ref[...] = x
    s = jnp.dot(x * x, ones3_ref[...], preferred_element_type=jnp.float32)
    inv = 1.0 / (jnp.sqrt(s) + 1e-6)
    xw = jnp.dot(x, WX_ref[...], preferred_element_type=jnp.float32)
    h = xw[:, :64] + xw[:, 64:128] * inv + b1_ref[...]
    v = jnp.dot(h * h, J64_ref[...], preferred_element_type=jnp.float32)
    a = _gelu(h * jax.lax.rsqrt(v + 1e-5) * g1_ref[...] + be1_ref[...])
    h = jnp.dot(a, W2_ref[...], preferred_element_type=jnp.float32) + b2_ref[...]
    v = jnp.dot(h * h, J128_ref[...], preferred_element_type=jnp.float32)
    a = _gelu(h * jax.lax.rsqrt(v + 1e-5) * g2_ref[...] + be2_ref[...])
    p = _gelu(xw[:, 128:] + pb1_ref[...])
    t = (jnp.dot(a, W3_ref[...], preferred_element_type=jnp.float32)
         + jnp.dot(p, P2_ref[...], preferred_element_type=jnp.float32)
         + b3p_ref[...])
    tok_ref[...] = t


def _full(shape):
    return pl.BlockSpec(shape, lambda i: (0,) * len(shape))


@jax.jit
def kernel(rgb, W1, b1, g1, be1, W2, b2, g2, be2, W3, b3, P1, pb1, P2, pb2):
    B, C, H, W = rgb.shape
    N = B * H * W
    x = jnp.transpose(rgb, (0, 2, 3, 1)).reshape(N, 3)

    C64 = jnp.eye(64, dtype=jnp.float32) - 1.0 / 64.0
    C128 = jnp.eye(128, dtype=jnp.float32) - 1.0 / 128.0
    WX = jnp.concatenate([W1[:3] @ C64, W1[3:] @ C64, P1], axis=1)
    b1c = (b1 - jnp.mean(b1)).reshape(1, -1)
    W2c = W2 @ C128
    b2c = (b2 - jnp.mean(b2)).reshape(1, -1)
    b3p = (b3 + pb2).reshape(1, -1)
    ones3 = jnp.ones((3, 64), jnp.float32)
    J64 = jnp.full((64, 64), 1.0 / 64.0, jnp.float32)
    J128 = jnp.full((128, 128), 1.0 / 128.0, jnp.float32)

    ws = [WX, b1c, g1.reshape(1, -1), be1.reshape(1, -1),
          W2c, b2c, g2.reshape(1, -1), be2.reshape(1, -1),
          W3, b3p, pb1.reshape(1, -1), P2, ones3, J64, J128]

    tok, pos = pl.pallas_call(
        _encoder_kernel,
        grid=(N // BLOCK_T,),
        in_specs=[pl.BlockSpec((BLOCK_T, 3), lambda i: (i, 0))]
                 + [_full(w.shape) for w in ws],
        out_specs=[
            pl.BlockSpec((BLOCK_T, OUT_D), lambda i: (i, 0)),
            pl.BlockSpec((BLOCK_T, 3), lambda i: (i, 0)),
        ],
        out_shape=[
            jax.ShapeDtypeStruct((N, OUT_D), jnp.float32),
            jax.ShapeDtypeStruct((N, 3), jnp.float32),
        ],
        compiler_params=pltpu.CompilerParams(
            dimension_semantics=("parallel",),
        ),
    )(x, *ws)

    tokens = tok.reshape(B, H * W, OUT_D)
    positions = pos.reshape(B, H * W, 3)
    return tokens, positions


# speedup vs baseline: 2.7129x; 1.1756x over previous
import jax
import jax.numpy as jnp
from jax.experimental import pallas as pl
from jax.experimental.pallas import tpu as pltpu

OUT_D = 256
BLOCK_T = 2048


def _gelu(x):
    return 0.5 * x * (1.0 + jax.lax.erf(x * 0.7071067811865476))


def _encoder_kernel(x_ref, WX_ref, b1_ref, g1_ref, be1_ref,
                    W2_ref, b2_ref, g2_ref, be2_ref, W3_ref, b3p_ref,
                    pb1_ref, P2_ref, ones3_ref, J64_ref, J128_ref,
                    tok_ref, pos_ref):
    x = x_ref[...] * 2.0 - 1.0
    pos_ref[...] = x
    s = jnp.dot(x * x, ones3_ref[...], preferred_element_type=jnp.float32)
    inv = 1.0 / (jnp.sqrt(s) + 1e-6)
    xw = jnp.dot(x, WX_ref[...], preferred_element_type=jnp.float32)
    h = xw[:, :64] + xw[:, 64:128] * inv + b1_ref[...]
    v = jnp.dot(h * h, J64_ref[...], preferred_element_type=jnp.float32)
    a = _gelu(h * jax.lax.rsqrt(v + 1e-5) * g1_ref[...] + be1_ref[...])
    h = jnp.dot(a, W2_ref[...], preferred_element_type=jnp.float32) + b2_ref[...]
    v = jnp.dot(h * h, J128_ref[...], preferred_element_type=jnp.float32)
    a = _gelu(h * jax.lax.rsqrt(v + 1e-5) * g2_ref[...] + be2_ref[...])
    p = _gelu(xw[:, 128:] + pb1_ref[...])
    t = (jnp.dot(a, W3_ref[...], preferred_element_type=jnp.float32)
         + jnp.dot(p, P2_ref[...], preferred_element_type=jnp.float32)
         + b3p_ref[...])
    tok_ref[...] = t


def _full(shape):
    return pl.BlockSpec(shape, lambda i: (0,) * len(shape))


@jax.jit
def kernel(rgb, W1, b1, g1, be1, W2, b2, g2, be2, W3, b3, P1, pb1, P2, pb2):
    B, C, H, W = rgb.shape
    N = B * H * W
    x = jnp.transpose(rgb, (0, 2, 3, 1)).reshape(N, 3)

    C64 = jnp.eye(64, dtype=jnp.float32) - 1.0 / 64.0
    C128 = jnp.eye(128, dtype=jnp.float32) - 1.0 / 128.0
    WX = jnp.concatenate([W1[:3] @ C64, W1[3:] @ C64, P1], axis=1)
    b1c = (b1 - jnp.mean(b1)).reshape(1, -1)
    W2c = W2 @ C128
    b2c = (b2 - jnp.mean(b2)).reshape(1, -1)
    b3p = (b3 + pb2).reshape(1, -1)
    ones3 = jnp.ones((3, 64), jnp.float32)
    J64 = jnp.full((64, 64), 1.0 / 64.0, jnp.float32)
    J128 = jnp.full((128, 128), 1.0 / 128.0, jnp.float32)

    ws = [WX, b1c, g1.reshape(1, -1), be1.reshape(1, -1),
          W2c, b2c, g2.reshape(1, -1), be2.reshape(1, -1),
          W3, b3p, pb1.reshape(1, -1), P2, ones3, J64, J128]

    tok, pos = pl.pallas_call(
        _encoder_kernel,
        grid=(N // BLOCK_T,),
        in_specs=[pl.BlockSpec((BLOCK_T, 3), lambda i: (i, 0))]
                 + [_full(w.shape) for w in ws],
        out_specs=[
            pl.BlockSpec((BLOCK_T, OUT_D), lambda i: (i, 0)),
            pl.BlockSpec((BLOCK_T, 3), lambda i: (i, 0)),
        ],
        out_shape=[
            jax.ShapeDtypeStruct((N, OUT_D), jnp.float32),
            jax.ShapeDtypeStruct((N, 3), jnp.float32),
        ],
        compiler_params=pltpu.CompilerParams(
            dimension_semantics=("parallel",),
        ),
    )(x, *ws)

    tokens = tok.reshape(B, H * W, OUT_D)
    positions = pos.reshape(B, H * W, 3)
    return tokens, positions


# speedup vs baseline: 2.8983x; 1.0684x over previous
import jax
import jax.numpy as jnp
from jax.experimental import pallas as pl
from jax.experimental.pallas import tpu as pltpu

OUT_D = 256
BLOCK_T = 4096


def _gelu(x):
    return 0.5 * x * (1.0 + jax.lax.erf(x * 0.7071067811865476))


def _encoder_kernel(x_ref, WX_ref, b1_ref, g1_ref, be1_ref,
                    W2_ref, b2_ref, g2_ref, be2_ref, W3_ref, b3p_ref,
                    pb1_ref, P2_ref, ones3_ref, J64_ref, J128_ref,
                    tok_ref, pos_ref):
    x = x_ref[...] * 2.0 - 1.0
    pos_ref[...] = x
    s = jnp.dot(x * x, ones3_ref[...], preferred_element_type=jnp.float32)
    inv = 1.0 / (jnp.sqrt(s) + 1e-6)
    xw = jnp.dot(x, WX_ref[...], preferred_element_type=jnp.float32)
    h = xw[:, :64] + xw[:, 64:128] * inv + b1_ref[...]
    v = jnp.dot(h * h, J64_ref[...], preferred_element_type=jnp.float32)
    a = _gelu(h * jax.lax.rsqrt(v + 1e-5) * g1_ref[...] + be1_ref[...])
    h = jnp.dot(a, W2_ref[...], preferred_element_type=jnp.float32) + b2_ref[...]
    v = jnp.dot(h * h, J128_ref[...], preferred_element_type=jnp.float32)
    a = _gelu(h * jax.lax.rsqrt(v + 1e-5) * g2_ref[...] + be2_ref[...])
    p = _gelu(xw[:, 128:] + pb1_ref[...])
    t = (jnp.dot(a, W3_ref[...], preferred_element_type=jnp.float32)
         + jnp.dot(p, P2_ref[...], preferred_element_type=jnp.float32)
         + b3p_ref[...])
    tok_ref[...] = t


def _full(shape):
    return pl.BlockSpec(shape, lambda i: (0,) * len(shape))


@jax.jit
def kernel(rgb, W1, b1, g1, be1, W2, b2, g2, be2, W3, b3, P1, pb1, P2, pb2):
    B, C, H, W = rgb.shape
    N = B * H * W
    x = jnp.transpose(rgb, (0, 2, 3, 1)).reshape(N, 3)

    C64 = jnp.eye(64, dtype=jnp.float32) - 1.0 / 64.0
    C128 = jnp.eye(128, dtype=jnp.float32) - 1.0 / 128.0
    WX = jnp.concatenate([W1[:3] @ C64, W1[3:] @ C64, P1], axis=1)
    b1c = (b1 - jnp.mean(b1)).reshape(1, -1)
    W2c = W2 @ C128
    b2c = (b2 - jnp.mean(b2)).reshape(1, -1)
    b3p = (b3 + pb2).reshape(1, -1)
    ones3 = jnp.ones((3, 64), jnp.float32)
    J64 = jnp.full((64, 64), 1.0 / 64.0, jnp.float32)
    J128 = jnp.full((128, 128), 1.0 / 128.0, jnp.float32)

    ws = [WX, b1c, g1.reshape(1, -1), be1.reshape(1, -1),
          W2c, b2c, g2.reshape(1, -1), be2.reshape(1, -1),
          W3, b3p, pb1.reshape(1, -1), P2, ones3, J64, J128]

    tok, pos = pl.pallas_call(
        _encoder_kernel,
        grid=(N // BLOCK_T,),
        in_specs=[pl.BlockSpec((BLOCK_T, 3), lambda i: (i, 0))]
                 + [_full(w.shape) for w in ws],
        out_specs=[
            pl.BlockSpec((BLOCK_T, OUT_D), lambda i: (i, 0)),
            pl.BlockSpec((BLOCK_T, 3), lambda i: (i, 0)),
        ],
        out_shape=[
            jax.ShapeDtypeStruct((N, OUT_D), jnp.float32),
            jax.ShapeDtypeStruct((N, 3), jnp.float32),
        ],
        compiler_params=pltpu.CompilerParams(
            dimension_semantics=("parallel",),
        ),
    )(x, *ws)

    tokens = tok.reshape(B, H * W, OUT_D)
    positions = pos.reshape(B, H * W, 3)
    return tokens, positions


# speedup vs baseline: 2.9356x; 1.0129x over previous
import jax
import jax.numpy as jnp
from jax.experimental import pallas as pl
from jax.experimental.pallas import tpu as pltpu

OUT_D = 256
BLOCK_T = 7168


def _gelu(x):
    return 0.5 * x * (1.0 + jax.lax.erf(x * 0.7071067811865476))


def _encoder_kernel(x_ref, WX_ref, b1_ref, g1_ref, be1_ref,
                    W2_ref, b2_ref, g2_ref, be2_ref, W3_ref, b3p_ref,
                    pb1_ref, P2_ref, ones3_ref, J64_ref, J128_ref,
                    tok_ref, pos_ref):
    x = x_ref[...] * 2.0 - 1.0
    pos_ref[...] = x
    s = jnp.dot(x * x, ones3_ref[...], preferred_element_type=jnp.float32)
    inv = 1.0 / (jnp.sqrt(s) + 1e-6)
    xw = jnp.dot(x, WX_ref[...], preferred_element_type=jnp.float32)
    h = xw[:, :64] + xw[:, 64:128] * inv + b1_ref[...]
    v = jnp.dot(h * h, J64_ref[...], preferred_element_type=jnp.float32)
    a = _gelu(h * jax.lax.rsqrt(v + 1e-5) * g1_ref[...] + be1_ref[...])
    h = jnp.dot(a, W2_ref[...], preferred_element_type=jnp.float32) + b2_ref[...]
    v = jnp.dot(h * h, J128_ref[...], preferred_element_type=jnp.float32)
    a = _gelu(h * jax.lax.rsqrt(v + 1e-5) * g2_ref[...] + be2_ref[...])
    p = _gelu(xw[:, 128:] + pb1_ref[...])
    t = (jnp.dot(a, W3_ref[...], preferred_element_type=jnp.float32)
         + jnp.dot(p, P2_ref[...], preferred_element_type=jnp.float32)
         + b3p_ref[...])
    tok_ref[...] = t


def _full(shape):
    return pl.BlockSpec(shape, lambda i: (0,) * len(shape))


@jax.jit
def kernel(rgb, W1, b1, g1, be1, W2, b2, g2, be2, W3, b3, P1, pb1, P2, pb2):
    B, C, H, W = rgb.shape
    N = B * H * W
    x = jnp.transpose(rgb, (0, 2, 3, 1)).reshape(N, 3)

    C64 = jnp.eye(64, dtype=jnp.float32) - 1.0 / 64.0
    C128 = jnp.eye(128, dtype=jnp.float32) - 1.0 / 128.0
    WX = jnp.concatenate([W1[:3] @ C64, W1[3:] @ C64, P1], axis=1)
    b1c = (b1 - jnp.mean(b1)).reshape(1, -1)
    W2c = W2 @ C128
    b2c = (b2 - jnp.mean(b2)).reshape(1, -1)
    b3p = (b3 + pb2).reshape(1, -1)
    ones3 = jnp.ones((3, 64), jnp.float32)
    J64 = jnp.full((64, 64), 1.0 / 64.0, jnp.float32)
    J128 = jnp.full((128, 128), 1.0 / 128.0, jnp.float32)

    ws = [WX, b1c, g1.reshape(1, -1), be1.reshape(1, -1),
          W2c, b2c, g2.reshape(1, -1), be2.reshape(1, -1),
          W3, b3p, pb1.reshape(1, -1), P2, ones3, J64, J128]

    tok, pos = pl.pallas_call(
        _encoder_kernel,
        grid=(N // BLOCK_T,),
        in_specs=[pl.BlockSpec((BLOCK_T, 3), lambda i: (i, 0))]
                 + [_full(w.shape) for w in ws],
        out_specs=[
            pl.BlockSpec((BLOCK_T, OUT_D), lambda i: (i, 0)),
            pl.BlockSpec((BLOCK_T, 3), lambda i: (i, 0)),
        ],
        out_shape=[
            jax.ShapeDtypeStruct((N, OUT_D), jnp.float32),
            jax.ShapeDtypeStruct((N, 3), jnp.float32),
        ],
        compiler_params=pltpu.CompilerParams(
            dimension_semantics=("parallel",),
        ),
    )(x, *ws)

    tokens = tok.reshape(B, H * W, OUT_D)
    positions = pos.reshape(B, H * W, 3)
    return tokens, positions


# speedup vs baseline: 3.2833x; 1.1184x over previous
import jax
import jax.numpy as jnp
from jax.experimental import pallas as pl
from jax.experimental.pallas import tpu as pltpu

OUT_D = 256
BLOCK_T = 3584


def _gelu(x):
    return 0.5 * x * (1.0 + jax.lax.erf(x * 0.7071067811865476))


def _dot0(a, b):
    return jax.lax.dot_general(a, b, (((0,), (0,)), ((), ())),
                               preferred_element_type=jnp.float32)


def _dot(a, b):
    return jnp.dot(a, b, preferred_element_type=jnp.float32)


def _encoder_kernel(r_ref, W1_ref, b1_ref, g1_ref, be1_ref,
                    W2_ref, b2_ref, g2_ref, be2_ref, W3_ref, b3_ref,
                    P1_ref, pb1_ref, P2_ref, pb2_ref,
                    eye3_ref, J64_ref, J128_ref,
                    tok_ref, pos_ref):
    xT = r_ref[0] * 2.0 - 1.0
    sT = jnp.sum(xT * xT, axis=0, keepdims=True)
    invT = 1.0 / (jnp.sqrt(sT) + 1e-6)
    nT = xT * invT
    fT = jnp.concatenate([xT, nT], axis=0)
    pos_ref[...] = _dot0(xT, eye3_ref[...])
    h = _dot0(fT, W1_ref[...]) + b1_ref[...]
    h = h - _dot(h, J64_ref[...])
    v = _dot(h * h, J64_ref[...])
    a = _gelu(h * jax.lax.rsqrt(v + 1e-5) * g1_ref[...] + be1_ref[...])
    h = _dot(a, W2_ref[...]) + b2_ref[...]
    h = h - _dot(h, J128_ref[...])
    v = _dot(h * h, J128_ref[...])
    a = _gelu(h * jax.lax.rsqrt(v + 1e-5) * g2_ref[...] + be2_ref[...])
    p = _gelu(_dot0(xT, P1_ref[...]) + pb1_ref[...])
    t = _dot(a, W3_ref[...]) + _dot(p, P2_ref[...])
    tok_ref[...] = t + (b3_ref[...] + pb2_ref[...])


def _full(shape):
    return pl.BlockSpec(shape, lambda b, i: (0,) * len(shape))


@jax.jit
def kernel(rgb, W1, b1, g1, be1, W2, b2, g2, be2, W3, b3, P1, pb1, P2, pb2):
    B, C, H, W = rgb.shape
    HW = H * W
    N = B * HW
    nblk = HW // BLOCK_T
    rgb3 = rgb.reshape(B, C, HW)

    eye3 = jnp.eye(3, dtype=jnp.float32)
    J64 = jnp.full((64, 64), 1.0 / 64.0, jnp.float32)
    J128 = jnp.full((128, 128), 1.0 / 128.0, jnp.float32)

    ws = [W1, b1.reshape(1, -1), g1.reshape(1, -1), be1.reshape(1, -1),
          W2, b2.reshape(1, -1), g2.reshape(1, -1), be2.reshape(1, -1),
          W3, b3.reshape(1, -1), P1, pb1.reshape(1, -1), P2,
          pb2.reshape(1, -1), eye3, J64, J128]

    tok, pos = pl.pallas_call(
        _encoder_kernel,
        grid=(B, nblk),
        in_specs=[pl.BlockSpec((1, C, BLOCK_T), lambda b, i: (b, 0, i))]
                 + [_full(w.shape) for w in ws],
        out_specs=[
            pl.BlockSpec((BLOCK_T, OUT_D), lambda b, i: (b * nblk + i, 0)),
            pl.BlockSpec((BLOCK_T, 3), lambda b, i: (b * nblk + i, 0)),
        ],
        out_shape=[
            jax.ShapeDtypeStruct((N, OUT_D), jnp.float32),
            jax.ShapeDtypeStruct((N, 3), jnp.float32),
        ],
        compiler_params=pltpu.CompilerParams(
            dimension_semantics=("parallel", "parallel"),
        ),
    )(rgb3, *ws)

    tokens = tok.reshape(B, HW, OUT_D)
    positions = pos.reshape(B, HW, 3)
    return tokens, positions


# speedup vs baseline: 3.3143x; 1.0094x over previous
import jax
import jax.numpy as jnp
from jax.experimental import pallas as pl
from jax.experimental.pallas import tpu as pltpu

OUT_D = 256
BLOCK_T = 3584


def _gelu(x):
    return 0.5 * x * (1.0 + jax.lax.erf(x * 0.7071067811865476))


def _dot0(a, b):
    return jax.lax.dot_general(a, b, (((0,), (0,)), ((), ())),
                               preferred_element_type=jnp.float32)


def _dot(a, b):
    return jnp.dot(a, b, preferred_element_type=jnp.float32)


def _encoder_kernel(r_ref, W1_ref, b1_ref, g1_ref, be1_ref,
                    W2_ref, b2_ref, g2_ref, be2_ref, W3_ref, b3_ref,
                    P1_ref, pb1_ref, P2_ref, pb2_ref,
                    eye3_ref, J64_ref, J128_ref,
                    tok_ref, pos_ref):
    rT = r_ref[0].reshape(3, -1)
    xT = rT * 2.0 - 1.0
    sT = jnp.sum(xT * xT, axis=0, keepdims=True)
    invT = 1.0 / (jnp.sqrt(sT) + 1e-6)
    nT = xT * invT
    fT = jnp.concatenate([xT, nT], axis=0)
    pos_ref[...] = _dot0(xT, eye3_ref[...])
    h = _dot0(fT, W1_ref[...]) + b1_ref[...]
    h = h - _dot(h, J64_ref[...])
    v = _dot(h * h, J64_ref[...])
    a = _gelu(h * jax.lax.rsqrt(v + 1e-5) * g1_ref[...] + be1_ref[...])
    h = _dot(a, W2_ref[...]) + b2_ref[...]
    h = h - _dot(h, J128_ref[...])
    v = _dot(h * h, J128_ref[...])
    a = _gelu(h * jax.lax.rsqrt(v + 1e-5) * g2_ref[...] + be2_ref[...])
    p = _gelu(_dot0(xT, P1_ref[...]) + pb1_ref[...])
    t = _dot(a, W3_ref[...]) + _dot(p, P2_ref[...])
    tok_ref[...] = t + (b3_ref[...] + pb2_ref[...])


def _full(shape):
    return pl.BlockSpec(shape, lambda b, i: (0,) * len(shape))


@jax.jit
def kernel(rgb, W1, b1, g1, be1, W2, b2, g2, be2, W3, b3, P1, pb1, P2, pb2):
    B, C, H, W = rgb.shape
    HW = H * W
    N = B * HW
    nblk = HW // BLOCK_T
    Hc = BLOCK_T // W

    eye3 = jnp.eye(3, dtype=jnp.float32)
    J64 = jnp.full((64, 64), 1.0 / 64.0, jnp.float32)
    J128 = jnp.full((128, 128), 1.0 / 128.0, jnp.float32)

    ws = [W1, b1.reshape(1, -1), g1.reshape(1, -1), be1.reshape(1, -1),
          W2, b2.reshape(1, -1), g2.reshape(1, -1), be2.reshape(1, -1),
          W3, b3.reshape(1, -1), P1, pb1.reshape(1, -1), P2,
          pb2.reshape(1, -1), eye3, J64, J128]

    tok, pos = pl.pallas_call(
        _encoder_kernel,
        grid=(B, nblk),
        in_specs=[pl.BlockSpec((1, C, Hc, W), lambda b, i: (b, 0, i, 0))]
                 + [_full(w.shape) for w in ws],
        out_specs=[
            pl.BlockSpec((BLOCK_T, OUT_D), lambda b, i: (b * nblk + i, 0)),
            pl.BlockSpec((BLOCK_T, 3), lambda b, i: (b * nblk + i, 0)),
        ],
        out_shape=[
            jax.ShapeDtypeStruct((N, OUT_D), jnp.float32),
            jax.ShapeDtypeStruct((N, 3), jnp.float32),
        ],
        compiler_params=pltpu.CompilerParams(
            dimension_semantics=("parallel", "parallel"),
        ),
    )(rgb, *ws)

    tokens = tok.reshape(B, HW, OUT_D)
    positions = pos.reshape(B, HW, 3)
    return tokens, positions
